# Initial kernel scaffold; baseline (speedup 1.0000x reference)
import functools

import jax
import jax.numpy as jnp
from jax import lax
from jax.experimental import pallas as pl
from jax.experimental.pallas import tpu as pltpu
from jax.experimental.pallas import tpu_sc as plsc

D = 32
B = 16384
L = 50
NW = 32
ROWS_W = B // NW
RPB = 2
IPB = RPB * L
NBLK = ROWS_W // RPB

_mesh = plsc.VectorSubcoreMesh(core_axis_name="c", subcore_axis_name="s")


@functools.partial(
    pl.kernel,
    out_type=jax.ShapeDtypeStruct((B, D), jnp.float32),
    mesh=_mesh,
    scratch_types=[
        pltpu.VMEM((NBLK, IPB), jnp.int32),
        pltpu.VMEM((ROWS_W, D), jnp.float32),
        pltpu.VMEM((IPB, D), jnp.float32),
        pltpu.SemaphoreType.DMA,
    ],
)
def _embed_sum(idx_hbm, table_hbm, out_hbm, idx_v, out_v, buf, sem):
    wid = lax.axis_index("c") * 16 + lax.axis_index("s")
    pltpu.sync_copy(idx_hbm.at[wid], idx_v)

    def body(j, carry):
        pltpu.async_copy(table_hbm.at[idx_v.at[j]], buf, sem).wait()
        for r in range(RPB):
            a0 = buf[r * L, pl.ds(0, 16)]
            a1 = buf[r * L, pl.ds(16, 16)]
            b0 = buf[r * L + 1, pl.ds(0, 16)]
            b1 = buf[r * L + 1, pl.ds(16, 16)]
            for l in range(2, L, 2):
                a0 = a0 + buf[r * L + l, pl.ds(0, 16)]
                a1 = a1 + buf[r * L + l, pl.ds(16, 16)]
                b0 = b0 + buf[r * L + l + 1, pl.ds(0, 16)]
                b1 = b1 + buf[r * L + l + 1, pl.ds(16, 16)]
            row = j * RPB + r
            out_v[row, pl.ds(0, 16)] = a0 + b0
            out_v[row, pl.ds(16, 16)] = a1 + b1
        return carry

    lax.fori_loop(0, NBLK, body, 0)
    pltpu.sync_copy(out_v, out_hbm.at[pl.ds(wid * ROWS_W, ROWS_W)])


def kernel(item_tensors, table):
    idx = item_tensors.astype(jnp.int32).reshape(NW, NBLK, IPB)
    return _embed_sum(idx, table)


# speedup vs baseline: 2.2568x; 2.2568x over previous
import functools

import jax
import jax.numpy as jnp
from jax import lax
from jax.experimental import pallas as pl
from jax.experimental.pallas import tpu as pltpu
from jax.experimental.pallas import tpu_sc as plsc

D = 32
B = 16384
L = 50
NW = 32
ROWS_W = B // NW
RPB = 2
IPB = RPB * L
NBLK = ROWS_W // RPB

_mesh = plsc.VectorSubcoreMesh(core_axis_name="c", subcore_axis_name="s")


@functools.partial(
    pl.kernel,
    out_type=jax.ShapeDtypeStruct((B, D), jnp.float32),
    mesh=_mesh,
    compiler_params=pltpu.CompilerParams(use_tc_tiling_on_sc=False),
    scratch_types=[
        pltpu.VMEM((NBLK, IPB), jnp.int32),
        pltpu.VMEM((ROWS_W, D), jnp.float32),
        pltpu.VMEM((IPB, D), jnp.float32),
        pltpu.SemaphoreType.DMA,
    ],
)
def _embed_sum(idx_hbm, table_hbm, out_hbm, idx_v, out_v, buf, sem):
    wid = lax.axis_index("c") * 16 + lax.axis_index("s")
    pltpu.sync_copy(idx_hbm.at[wid], idx_v)

    def body(j, carry):
        pltpu.async_copy(table_hbm.at[idx_v.at[j]], buf, sem).wait()
        for r in range(RPB):
            a0 = buf[r * L, pl.ds(0, 16)]
            a1 = buf[r * L, pl.ds(16, 16)]
            b0 = buf[r * L + 1, pl.ds(0, 16)]
            b1 = buf[r * L + 1, pl.ds(16, 16)]
            for l in range(2, L, 2):
                a0 = a0 + buf[r * L + l, pl.ds(0, 16)]
                a1 = a1 + buf[r * L + l, pl.ds(16, 16)]
                b0 = b0 + buf[r * L + l + 1, pl.ds(0, 16)]
                b1 = b1 + buf[r * L + l + 1, pl.ds(16, 16)]
            row = j * RPB + r
            out_v[row, pl.ds(0, 16)] = a0 + b0
            out_v[row, pl.ds(16, 16)] = a1 + b1
        return carry

    lax.fori_loop(0, NBLK, body, 0)
    pltpu.sync_copy(out_v, out_hbm.at[pl.ds(wid * ROWS_W, ROWS_W)])


def kernel(item_tensors, table):
    idx = item_tensors.astype(jnp.int32).reshape(NW, NBLK, IPB)
    return _embed_sum(idx, table)


# speedup vs baseline: 2.8567x; 1.2658x over previous
import functools

import jax
import jax.numpy as jnp
from jax import lax
from jax.experimental import pallas as pl
from jax.experimental.pallas import tpu as pltpu
from jax.experimental.pallas import tpu_sc as plsc

D = 32
B = 16384
L = 50
NW = 32
ROWS_W = B // NW
RPB = 2
IPB = RPB * L
NBLK = ROWS_W // RPB
NBUF = 4

_mesh = plsc.VectorSubcoreMesh(core_axis_name="c", subcore_axis_name="s")


@functools.partial(
    pl.kernel,
    out_type=jax.ShapeDtypeStruct((B, D), jnp.float32),
    mesh=_mesh,
    compiler_params=pltpu.CompilerParams(use_tc_tiling_on_sc=False),
    scratch_types=[
        pltpu.VMEM((NBLK, IPB), jnp.int32),
        pltpu.VMEM((ROWS_W, D), jnp.float32),
        pltpu.VMEM((NBUF, IPB, D), jnp.float32),
        pltpu.SemaphoreType.DMA,
        pltpu.SemaphoreType.DMA,
        pltpu.SemaphoreType.DMA,
        pltpu.SemaphoreType.DMA,
    ],
)
def _embed_sum(idx_hbm, table_hbm, out_hbm, idx_v, out_v, bufs, s0, s1, s2, s3):
    wid = lax.axis_index("c") * 16 + lax.axis_index("s")
    sems = (s0, s1, s2, s3)
    pltpu.sync_copy(idx_hbm.at[wid], idx_v)

    def start(blk, b):
        return pltpu.async_copy(table_hbm.at[idx_v.at[blk]], bufs.at[b], sems[b])

    def accum(blk, b):
        buf = bufs.at[b]
        pltpu.make_async_copy(table_hbm.at[idx_v.at[blk]], buf, sems[b]).wait()
        for r in range(RPB):
            a0 = buf[r * L, pl.ds(0, 16)]
            a1 = buf[r * L, pl.ds(16, 16)]
            b0 = buf[r * L + 1, pl.ds(0, 16)]
            b1 = buf[r * L + 1, pl.ds(16, 16)]
            for l in range(2, L, 2):
                a0 = a0 + buf[r * L + l, pl.ds(0, 16)]
                a1 = a1 + buf[r * L + l, pl.ds(16, 16)]
                b0 = b0 + buf[r * L + l + 1, pl.ds(0, 16)]
                b1 = b1 + buf[r * L + l + 1, pl.ds(16, 16)]
            row = blk * RPB + r
            out_v[row, pl.ds(0, 16)] = a0 + b0
            out_v[row, pl.ds(16, 16)] = a1 + b1

    for b in range(NBUF):
        start(b, b)

    def body(j, carry):
        for b in range(NBUF):
            accum(j + b, b)
            start(j + b + NBUF, b)
        return carry

    lax.fori_loop(0, (NBLK - NBUF) // NBUF, lambda i, c: body(i * NBUF, c), 0,
                  unroll=False)
    for b in range(NBUF):
        accum(NBLK - NBUF + b, b)
    pltpu.sync_copy(out_v, out_hbm.at[pl.ds(wid * ROWS_W, ROWS_W)])


def kernel(item_tensors, table):
    idx = item_tensors.astype(jnp.int32).reshape(NW, NBLK, IPB)
    return _embed_sum(idx, table)
